# baseline (speedup 1.0000x reference)
import jax
import jax.numpy as jnp
from jax import lax
from jax.experimental import pallas as pl
from jax.experimental.pallas import tpu as pltpu

B = 16
L = 4096
D = 512
RB = 16
RBS = L // RB


def _tc_body(mask_ref, emb_ref, out_ref, mx_ref, mn_ref):
    rb = pl.program_id(1)
    e = emb_ref[0]
    m = mask_ref[0] == 1
    inf = jnp.float32(jnp.inf)
    mx = jnp.max(jnp.where(m, e, -inf), axis=0, keepdims=True)
    mn = jnp.min(jnp.where(m, e, inf), axis=0, keepdims=True)

    @pl.when(rb == 0)
    def _():
        mx_ref[...] = mx
        mn_ref[...] = mn

    @pl.when(rb > 0)
    def _():
        mx_ref[...] = jnp.maximum(mx_ref[...], mx)
        mn_ref[...] = jnp.minimum(mn_ref[...], mn)

    @pl.when(rb == RB - 1)
    def _():
        out_ref[0] = mx_ref[...] - mn_ref[...]


@jax.jit
def _run_tc(embeddings, mask32):
    return pl.pallas_call(
        _tc_body,
        grid=(B, RB),
        in_specs=[
            pl.BlockSpec((1, RBS, 1), lambda b, rb: (b, rb, 0)),
            pl.BlockSpec((1, RBS, D), lambda b, rb: (b, rb, 0)),
        ],
        out_specs=pl.BlockSpec((1, 1, D), lambda b, rb: (b, 0, 0)),
        out_shape=jax.ShapeDtypeStruct((B, 1, D), jnp.float32),
        scratch_shapes=[
            pltpu.VMEM((1, D), jnp.float32),
            pltpu.VMEM((1, D), jnp.float32),
        ],
        compiler_params=pltpu.CompilerParams(
            dimension_semantics=("arbitrary", "arbitrary"),
        ),
    )(mask32.reshape(B, L, 1), embeddings).reshape(B, D)


def kernel(embeddings, mask):
    return _run_tc(embeddings, mask.astype(jnp.int32))


# speedup vs baseline: 1.9307x; 1.9307x over previous
import jax
import jax.numpy as jnp
from jax import lax
from jax.experimental import pallas as pl
from jax.experimental.pallas import tpu as pltpu

B = 16
L = 4096
D = 512
RB = 4
RBS = L // RB


def _tc_body(mask_ref, emb_ref, out_ref, mx_ref, mn_ref):
    rb = pl.program_id(1)
    e = emb_ref[0]
    m = mask_ref[0] == 1
    inf = jnp.float32(jnp.inf)
    mx = jnp.max(jnp.where(m, e, -inf), axis=0, keepdims=True)
    mn = jnp.min(jnp.where(m, e, inf), axis=0, keepdims=True)

    @pl.when(rb == 0)
    def _():
        mx_ref[...] = mx
        mn_ref[...] = mn

    @pl.when(rb > 0)
    def _():
        mx_ref[...] = jnp.maximum(mx_ref[...], mx)
        mn_ref[...] = jnp.minimum(mn_ref[...], mn)

    @pl.when(rb == RB - 1)
    def _():
        out_ref[0] = mx_ref[...] - mn_ref[...]


@jax.jit
def _run_tc(embeddings, mask32):
    return pl.pallas_call(
        _tc_body,
        grid=(B, RB),
        in_specs=[
            pl.BlockSpec((1, RBS, 1), lambda b, rb: (b, rb, 0)),
            pl.BlockSpec((1, RBS, D), lambda b, rb: (b, rb, 0)),
        ],
        out_specs=pl.BlockSpec((1, 1, D), lambda b, rb: (b, 0, 0)),
        out_shape=jax.ShapeDtypeStruct((B, 1, D), jnp.float32),
        scratch_shapes=[
            pltpu.VMEM((1, D), jnp.float32),
            pltpu.VMEM((1, D), jnp.float32),
        ],
        compiler_params=pltpu.CompilerParams(
            dimension_semantics=("arbitrary", "arbitrary"),
        ),
    )(mask32.reshape(B, L, 1), embeddings).reshape(B, D)


def kernel(embeddings, mask):
    return _run_tc(embeddings, mask.astype(jnp.int32))


# speedup vs baseline: 2.5689x; 1.3305x over previous
import jax
import jax.numpy as jnp
from jax import lax
from jax.experimental import pallas as pl
from jax.experimental.pallas import tpu as pltpu

B = 16
L = 4096
D = 512
RB = 4
RBS = L // RB


def _tc_body(mask_ref, emb_ref, out_ref):
    e = emb_ref[0]
    m = mask_ref[0] == 1
    inf = jnp.float32(jnp.inf)
    mx = jnp.max(jnp.where(m, e, -inf), axis=0, keepdims=True)
    mn = jnp.min(jnp.where(m, e, inf), axis=0, keepdims=True)
    out_ref[0] = mx - mn


@jax.jit
def _run_tc(embeddings, mask32):
    return pl.pallas_call(
        _tc_body,
        grid=(B,),
        in_specs=[
            pl.BlockSpec((1, L, 1), lambda b: (b, 0, 0)),
            pl.BlockSpec((1, L, D), lambda b: (b, 0, 0)),
        ],
        out_specs=pl.BlockSpec((1, 1, D), lambda b: (b, 0, 0)),
        out_shape=jax.ShapeDtypeStruct((B, 1, D), jnp.float32),
        compiler_params=pltpu.CompilerParams(
            dimension_semantics=("arbitrary",),
        ),
    )(mask32.reshape(B, L, 1), embeddings).reshape(B, D)


def kernel(embeddings, mask):
    return _run_tc(embeddings, mask.astype(jnp.int32))


# speedup vs baseline: 2.6669x; 1.0382x over previous
import jax
import jax.numpy as jnp
from jax import lax
from jax.experimental import pallas as pl
from jax.experimental.pallas import tpu as pltpu

B = 16
L = 4096
D = 512
RB = 4
RBS = L // RB


def _tc_body(mask_ref, emb_ref, out_ref):
    e = emb_ref[0]
    m = mask_ref[0] == 1
    inf = jnp.float32(jnp.inf)
    mx = jnp.max(jnp.where(m, e, -inf), axis=0, keepdims=True)
    mn = jnp.min(jnp.where(m, e, inf), axis=0, keepdims=True)
    out_ref[0] = mx - mn


@jax.jit
def _run_tc(embeddings, mask32):
    return pl.pallas_call(
        _tc_body,
        grid=(B,),
        in_specs=[
            pl.BlockSpec((1, L, 1), lambda b: (b, 0, 0)),
            pl.BlockSpec((1, L, D), lambda b: (b, 0, 0)),
        ],
        out_specs=pl.BlockSpec((1, 1, D), lambda b: (b, 0, 0)),
        out_shape=jax.ShapeDtypeStruct((B, 1, D), jnp.float32),
        compiler_params=pltpu.CompilerParams(
            dimension_semantics=("parallel",),
        ),
    )(mask32.reshape(B, L, 1), embeddings).reshape(B, D)


def kernel(embeddings, mask):
    return _run_tc(embeddings, mask.astype(jnp.int32))
